# Initial kernel scaffold; baseline (speedup 1.0000x reference)
import jax
import jax.numpy as jnp
from jax.experimental import pallas as pl


def kernel(encoded_tokens, pos_table):
    raise NotImplementedError("write your pallas kernel here")


# speedup vs baseline: 1.0108x; 1.0108x over previous
import jax
import jax.numpy as jnp
from jax.experimental import pallas as pl
from jax.experimental.pallas import tpu as pltpu


def _body(tok_ref, tab_ref, out_ref):
    out_ref[...] = tok_ref[...] + tab_ref[...]


def kernel(encoded_tokens, pos_table):
    B, T, D = encoded_tokens.shape
    BT = 512

    return pl.pallas_call(
        _body,
        grid=(T // BT, B),
        in_specs=[
            pl.BlockSpec((1, BT, D), lambda t, b: (b, t, 0)),
            pl.BlockSpec((BT, D), lambda t, b: (t, 0)),
        ],
        out_specs=pl.BlockSpec((1, BT, D), lambda t, b: (b, t, 0)),
        out_shape=jax.ShapeDtypeStruct((B, T, D), encoded_tokens.dtype),
        compiler_params=pltpu.CompilerParams(
            dimension_semantics=("arbitrary", "arbitrary"),
        ),
    )(encoded_tokens, pos_table)


# speedup vs baseline: 2.2008x; 2.1772x over previous
import jax
import jax.numpy as jnp
from jax.experimental import pallas as pl
from jax.experimental.pallas import tpu as pltpu


def _body(tok_ref, tab_ref, out_ref):
    out_ref[...] = tok_ref[...] + tab_ref[...]


def kernel(encoded_tokens, pos_table):
    B, T, D = encoded_tokens.shape
    BT = 2048

    return pl.pallas_call(
        _body,
        grid=(T // BT, B),
        in_specs=[
            pl.BlockSpec((1, BT, D), lambda t, b: (b, t, 0)),
            pl.BlockSpec((BT, D), lambda t, b: (t, 0)),
        ],
        out_specs=pl.BlockSpec((1, BT, D), lambda t, b: (b, t, 0)),
        out_shape=jax.ShapeDtypeStruct((B, T, D), encoded_tokens.dtype),
        compiler_params=pltpu.CompilerParams(
            dimension_semantics=("arbitrary", "arbitrary"),
        ),
    )(encoded_tokens, pos_table)


# speedup vs baseline: 2.7951x; 1.2700x over previous
import jax
import jax.numpy as jnp
from jax.experimental import pallas as pl
from jax.experimental.pallas import tpu as pltpu


def _body(tok_ref, tab_ref, out_ref):
    out_ref[...] = tok_ref[...] + tab_ref[...]


def kernel(encoded_tokens, pos_table):
    B, T, D = encoded_tokens.shape
    BT = 4096

    return pl.pallas_call(
        _body,
        grid=(T // BT, B),
        in_specs=[
            pl.BlockSpec((1, BT, D), lambda t, b: (b, t, 0)),
            pl.BlockSpec((BT, D), lambda t, b: (t, 0)),
        ],
        out_specs=pl.BlockSpec((1, BT, D), lambda t, b: (b, t, 0)),
        out_shape=jax.ShapeDtypeStruct((B, T, D), encoded_tokens.dtype),
        compiler_params=pltpu.CompilerParams(
            dimension_semantics=("arbitrary", "arbitrary"),
        ),
    )(encoded_tokens, pos_table)


# speedup vs baseline: 3.0584x; 1.0942x over previous
import jax
import jax.numpy as jnp
from jax.experimental import pallas as pl
from jax.experimental.pallas import tpu as pltpu


def _body(tok_ref, tab_ref, out_ref):
    out_ref[...] = tok_ref[...] + tab_ref[...]


def kernel(encoded_tokens, pos_table):
    B, T, D = encoded_tokens.shape
    BT = 8192

    return pl.pallas_call(
        _body,
        grid=(T // BT, B),
        in_specs=[
            pl.BlockSpec((1, BT, D), lambda t, b: (b, t, 0)),
            pl.BlockSpec((BT, D), lambda t, b: (t, 0)),
        ],
        out_specs=pl.BlockSpec((1, BT, D), lambda t, b: (b, t, 0)),
        out_shape=jax.ShapeDtypeStruct((B, T, D), encoded_tokens.dtype),
        compiler_params=pltpu.CompilerParams(
            dimension_semantics=("arbitrary", "arbitrary"),
        ),
    )(encoded_tokens, pos_table)


# speedup vs baseline: 3.4527x; 1.1289x over previous
import jax
import jax.numpy as jnp
from jax.experimental import pallas as pl
from jax.experimental.pallas import tpu as pltpu


def _body(tok_ref, tab_ref, out_ref):
    out_ref[...] = tok_ref[...] + tab_ref[...]


def kernel(encoded_tokens, pos_table):
    B, T, D = encoded_tokens.shape
    BT = 8192
    BB = 2

    return pl.pallas_call(
        _body,
        grid=(T // BT, B // BB),
        in_specs=[
            pl.BlockSpec((BB, BT, D), lambda t, b: (b, t, 0)),
            pl.BlockSpec((BT, D), lambda t, b: (t, 0)),
        ],
        out_specs=pl.BlockSpec((BB, BT, D), lambda t, b: (b, t, 0)),
        out_shape=jax.ShapeDtypeStruct((B, T, D), encoded_tokens.dtype),
        compiler_params=pltpu.CompilerParams(
            dimension_semantics=("arbitrary", "arbitrary"),
        ),
    )(encoded_tokens, pos_table)
